# baseline (speedup 1.0000x reference)
import jax
import jax.numpy as jnp
from jax.experimental import pallas as pl
from jax.experimental.pallas import tpu as pltpu
from jax.experimental.pallas import tpu_sc as plsc

SOM_H = 64
SOM_W = 128
K = SOM_H * SOM_W
Z = 256


def _esq_body(cb_ref, esq_ref):
    cb = cb_ref[...]
    col = jnp.sum(cb * cb, axis=1, keepdims=True)
    esq_ref[...] = jnp.transpose(col)


def _main_body(inv_ref, x_ref, wenc_ref, benc_ref, cb_ref, esq_ref,
               wdc_ref, bdc_ref, wdd_ref, bdd_ref,
               z_ref, dist_ref, idxrow_ref, xhc_ref, xhd_ref,
               wgt_ref, nb_ref):
    xb = x_ref[...].reshape(x_ref.shape[0], -1)
    z = jnp.dot(xb.astype(jnp.bfloat16),
                wenc_ref[...].astype(jnp.bfloat16),
                preferred_element_type=jnp.float32) + benc_ref[...]
    z = jnp.maximum(z, 0.0)
    z_ref[...] = z

    zcb = jax.lax.dot_general(z.astype(jnp.bfloat16),
                              cb_ref[...].astype(jnp.bfloat16),
                              (((1,), (1,)), ((), ())),
                              preferred_element_type=jnp.float32)
    zsq = jnp.sum(z * z, axis=1, keepdims=True)
    dist = (zsq - 2.0 * zcb) + esq_ref[...]
    dist_ref[...] = dist

    minval = jnp.min(dist, axis=1, keepdims=True)
    lane = jax.lax.broadcasted_iota(jnp.int32, dist.shape, 1)
    idx = jnp.min(jnp.where(dist == minval, lane, K), axis=1, keepdims=True)
    idxrow_ref[...] = jnp.transpose(idx)[None, :, :]

    zb = z.astype(jnp.bfloat16)
    xhc = jnp.dot(zb, wdc_ref[...].astype(jnp.bfloat16),
                  preferred_element_type=jnp.float32) + bdc_ref[...]
    xhc_ref[...] = xhc.reshape(xhc_ref.shape)
    xhd = jnp.dot(zb, wdd_ref[...].astype(jnp.bfloat16),
                  preferred_element_type=jnp.float32) + bdd_ref[...]
    xhd_ref[...] = xhd.reshape(xhd_ref.shape)

    r = idx // SOM_W
    c = idx % SOM_W
    up = jnp.clip(r - 1, 0, SOM_H - 1) * SOM_W + c
    down = jnp.clip(r + 1, 0, SOM_H - 1) * SOM_W + c
    left = r * SOM_W + jnp.clip(c - 1, 0, SOM_W - 1)
    right = r * SOM_W + jnp.clip(c + 1, 0, SOM_W - 1)
    nb_ref[...] = jnp.concatenate([idx, up, down, left, right], axis=1)

    inv = inv_ref[...]
    rows = z.shape[0]
    rf = r.astype(jnp.float32)
    cf = c.astype(jnp.float32)
    rowg = jax.lax.broadcasted_iota(jnp.int32, (rows, SOM_H), 1).astype(jnp.float32)
    colg = jax.lax.broadcasted_iota(jnp.int32, (rows, SOM_W), 1).astype(jnp.float32)
    rowexp = jnp.exp(-((rowg - rf) ** 2) * inv)
    colexp = jnp.exp(-((colg - cf) ** 2) * inv)
    wgt_ref[...] = (rowexp[:, :, None] * colexp[:, None, :]).reshape(rows, K)


def _sc_gather_rows(table, idx_row):
    n = idx_row.shape[1]
    win = 128
    mesh = plsc.VectorSubcoreMesh(core_axis_name="core",
                                  subcore_axis_name="subcore")

    @pl.kernel(out_type=jax.ShapeDtypeStruct((n, table.shape[1]), table.dtype),
               mesh=mesh)
    def gather_kernel(tab_hbm, i_hbm, o_hbm):
        def body(i_vmem, o_vmem):
            pltpu.sync_copy(tab_hbm.at[i_vmem.at[0]], o_vmem)

        pltpu.emit_pipeline(
            body,
            grid=(n // win,),
            in_specs=[pl.BlockSpec((1, win), index_map=lambda i: (0, i))],
            out_specs=[pl.BlockSpec((win, table.shape[1]),
                                    index_map=lambda i: (i, 0))],
            core_axis_name=("core", "subcore"),
            dimension_semantics=(pltpu.PARALLEL,),
        )(i_hbm, o_hbm)

    return gather_kernel(table, idx_row)


def kernel(x, epoch, W_enc, b_enc, codebook,
           W_dec_cont, b_dec_cont, W_dec_disc, b_dec_disc):
    B = x.shape[0]
    C, T = x.shape[1], x.shape[2]
    CT = C * T

    esq = pl.pallas_call(
        _esq_body,
        grid=(1,),
        in_specs=[pl.BlockSpec((K, Z), lambda i: (0, 0))],
        out_specs=pl.BlockSpec((1, K), lambda i: (0, 0)),
        out_shape=jax.ShapeDtypeStruct((1, K), jnp.float32),
    )(codebook)

    sigma = jnp.maximum(0.5, 10.0 * jnp.exp(-jnp.asarray(epoch, jnp.float32) / 20.0))
    inv2s = (1.0 / (2.0 * sigma * sigma)).reshape(1, 1)

    BB = 64
    z_cont, dist, idxrow, xhc, xhd, wgt, nb = pl.pallas_call(
        _main_body,
        grid=(B // BB,),
        in_specs=[
            pl.BlockSpec((1, 1), lambda i: (0, 0)),
            pl.BlockSpec((BB, C, T), lambda i: (i, 0, 0)),
            pl.BlockSpec((CT, Z), lambda i: (0, 0)),
            pl.BlockSpec((1, Z), lambda i: (0, 0)),
            pl.BlockSpec((K, Z), lambda i: (0, 0)),
            pl.BlockSpec((1, K), lambda i: (0, 0)),
            pl.BlockSpec((Z, CT), lambda i: (0, 0)),
            pl.BlockSpec((1, CT), lambda i: (0, 0)),
            pl.BlockSpec((Z, CT), lambda i: (0, 0)),
            pl.BlockSpec((1, CT), lambda i: (0, 0)),
        ],
        out_specs=[
            pl.BlockSpec((BB, Z), lambda i: (i, 0)),
            pl.BlockSpec((BB, K), lambda i: (i, 0)),
            pl.BlockSpec((1, 1, BB), lambda i: (i, 0, 0)),
            pl.BlockSpec((BB, C, T), lambda i: (i, 0, 0)),
            pl.BlockSpec((BB, C, T), lambda i: (i, 0, 0)),
            pl.BlockSpec((BB, K), lambda i: (i, 0)),
            pl.BlockSpec((BB, 5), lambda i: (i, 0)),
        ],
        out_shape=[
            jax.ShapeDtypeStruct((B, Z), jnp.float32),
            jax.ShapeDtypeStruct((B, K), jnp.float32),
            jax.ShapeDtypeStruct((B // BB, 1, BB), jnp.int32),
            jax.ShapeDtypeStruct((B, C, T), jnp.float32),
            jax.ShapeDtypeStruct((B, C, T), jnp.float32),
            jax.ShapeDtypeStruct((B, K), jnp.float32),
            jax.ShapeDtypeStruct((B, 5), jnp.int32),
        ],
        compiler_params=pltpu.CompilerParams(
            dimension_semantics=("parallel",)),
    )(inv2s, x, W_enc, b_enc.reshape(1, Z), codebook, esq,
      W_dec_cont, b_dec_cont.reshape(1, CT),
      W_dec_disc, b_dec_disc.reshape(1, CT))

    z_disc = _sc_gather_rows(codebook, idxrow.reshape(1, B))

    codebook_idxs = idxrow.reshape(B)
    return (codebook_idxs, z_cont, z_disc, nb, wgt, dist, xhc, xhd)


# speedup vs baseline: 1.1656x; 1.1656x over previous
import jax
import jax.numpy as jnp
from jax.experimental import pallas as pl
from jax.experimental.pallas import tpu as pltpu
from jax.experimental.pallas import tpu_sc as plsc

SOM_H = 64
SOM_W = 128
K = SOM_H * SOM_W
Z = 256


def _esq_body(cb_ref, esq_ref):
    cb = cb_ref[...]
    col = jnp.sum(cb * cb, axis=1, keepdims=True)
    esq_ref[...] = jnp.transpose(col)


def _encq_body(x_ref, wenc_ref, benc_ref, cb_ref, esq_ref,
               z_ref, dist_ref, idx_ref, idxrow_ref):
    xb = x_ref[...].reshape(x_ref.shape[0], -1)
    z = jnp.dot(xb.astype(jnp.bfloat16),
                wenc_ref[...].astype(jnp.bfloat16),
                preferred_element_type=jnp.float32) + benc_ref[...]
    z = jnp.maximum(z, 0.0)
    z_ref[...] = z

    zcb = jax.lax.dot_general(z.astype(jnp.bfloat16),
                              cb_ref[...].astype(jnp.bfloat16),
                              (((1,), (1,)), ((), ())),
                              preferred_element_type=jnp.float32)
    zsq = jnp.sum(z * z, axis=1, keepdims=True)
    dist = (zsq - 2.0 * zcb) + esq_ref[...]
    dist_ref[...] = dist

    minval = jnp.min(dist, axis=1, keepdims=True)
    lane = jax.lax.broadcasted_iota(jnp.int32, dist.shape, 1)
    idx = jnp.min(jnp.where(dist == minval, lane, K), axis=1, keepdims=True)
    idx_ref[...] = idx
    idxrow_ref[...] = jnp.transpose(idx)[None, :, :]


def _dec_body(inv_ref, z_ref, idx_ref, wdc_ref, bdc_ref, wdd_ref, bdd_ref,
              xhc_ref, xhd_ref, wgt_ref, nb_ref):
    z = z_ref[...]
    idx = idx_ref[...]
    zb = z.astype(jnp.bfloat16)
    xhc = jnp.dot(zb, wdc_ref[...].astype(jnp.bfloat16),
                  preferred_element_type=jnp.float32) + bdc_ref[...]
    xhc_ref[...] = xhc.reshape(xhc_ref.shape)
    xhd = jnp.dot(zb, wdd_ref[...].astype(jnp.bfloat16),
                  preferred_element_type=jnp.float32) + bdd_ref[...]
    xhd_ref[...] = xhd.reshape(xhd_ref.shape)

    r = idx // SOM_W
    c = idx % SOM_W
    up = jnp.clip(r - 1, 0, SOM_H - 1) * SOM_W + c
    down = jnp.clip(r + 1, 0, SOM_H - 1) * SOM_W + c
    left = r * SOM_W + jnp.clip(c - 1, 0, SOM_W - 1)
    right = r * SOM_W + jnp.clip(c + 1, 0, SOM_W - 1)
    nb_ref[...] = jnp.concatenate([idx, up, down, left, right], axis=1)

    inv = inv_ref[...]
    rows = z.shape[0]
    rf = r.astype(jnp.float32)
    cf = c.astype(jnp.float32)
    rowg = jax.lax.broadcasted_iota(jnp.int32, (rows, SOM_H), 1).astype(jnp.float32)
    colg = jax.lax.broadcasted_iota(jnp.int32, (rows, SOM_W), 1).astype(jnp.float32)
    rowexp = jnp.exp(-((rowg - rf) ** 2) * inv)
    colexp = jnp.exp(-((colg - cf) ** 2) * inv)
    wgt_ref[...] = (rowexp[:, :, None] * colexp[:, None, :]).reshape(rows, K)


def _sc_gather_rows(table, idx_row):
    n = idx_row.shape[1]
    win = 128
    mesh = plsc.VectorSubcoreMesh(core_axis_name="core",
                                  subcore_axis_name="subcore")

    @pl.kernel(out_type=jax.ShapeDtypeStruct((n, table.shape[1]), table.dtype),
               mesh=mesh)
    def gather_kernel(tab_hbm, i_hbm, o_hbm):
        def body(i_vmem, o_vmem):
            pltpu.sync_copy(tab_hbm.at[i_vmem.at[0]], o_vmem)

        pltpu.emit_pipeline(
            body,
            grid=(n // win,),
            in_specs=[pl.BlockSpec((1, win), index_map=lambda i: (0, i))],
            out_specs=[pl.BlockSpec((win, table.shape[1]),
                                    index_map=lambda i: (i, 0))],
            core_axis_name=("core", "subcore"),
            dimension_semantics=(pltpu.PARALLEL,),
        )(i_hbm, o_hbm)

    return gather_kernel(table, idx_row)


def kernel(x, epoch, W_enc, b_enc, codebook,
           W_dec_cont, b_dec_cont, W_dec_disc, b_dec_disc):
    B = x.shape[0]
    C, T = x.shape[1], x.shape[2]
    CT = C * T

    esq = pl.pallas_call(
        _esq_body,
        grid=(1,),
        in_specs=[pl.BlockSpec((K, Z), lambda i: (0, 0))],
        out_specs=pl.BlockSpec((1, K), lambda i: (0, 0)),
        out_shape=jax.ShapeDtypeStruct((1, K), jnp.float32),
    )(codebook)

    BA = 128
    z_cont, dist, idx2d, idxrow = pl.pallas_call(
        _encq_body,
        grid=(B // BA,),
        in_specs=[
            pl.BlockSpec((BA, C, T), lambda i: (i, 0, 0)),
            pl.BlockSpec((CT, Z), lambda i: (0, 0)),
            pl.BlockSpec((1, Z), lambda i: (0, 0)),
            pl.BlockSpec((K, Z), lambda i: (0, 0)),
            pl.BlockSpec((1, K), lambda i: (0, 0)),
        ],
        out_specs=[
            pl.BlockSpec((BA, Z), lambda i: (i, 0)),
            pl.BlockSpec((BA, K), lambda i: (i, 0)),
            pl.BlockSpec((BA, 1), lambda i: (i, 0)),
            pl.BlockSpec((1, 1, BA), lambda i: (i, 0, 0)),
        ],
        out_shape=[
            jax.ShapeDtypeStruct((B, Z), jnp.float32),
            jax.ShapeDtypeStruct((B, K), jnp.float32),
            jax.ShapeDtypeStruct((B, 1), jnp.int32),
            jax.ShapeDtypeStruct((B // BA, 1, BA), jnp.int32),
        ],
        compiler_params=pltpu.CompilerParams(
            dimension_semantics=("parallel",)),
    )(x, W_enc, b_enc.reshape(1, Z), codebook, esq)

    z_disc = _sc_gather_rows(codebook, idxrow.reshape(1, B))

    sigma = jnp.maximum(0.5, 10.0 * jnp.exp(-jnp.asarray(epoch, jnp.float32) / 20.0))
    inv2s = (1.0 / (2.0 * sigma * sigma)).reshape(1, 1)

    BB = 128
    xhc, xhd, wgt, nb = pl.pallas_call(
        _dec_body,
        grid=(B // BB,),
        in_specs=[
            pl.BlockSpec((1, 1), lambda i: (0, 0)),
            pl.BlockSpec((BB, Z), lambda i: (i, 0)),
            pl.BlockSpec((BB, 1), lambda i: (i, 0)),
            pl.BlockSpec((Z, CT), lambda i: (0, 0)),
            pl.BlockSpec((1, CT), lambda i: (0, 0)),
            pl.BlockSpec((Z, CT), lambda i: (0, 0)),
            pl.BlockSpec((1, CT), lambda i: (0, 0)),
        ],
        out_specs=[
            pl.BlockSpec((BB, C, T), lambda i: (i, 0, 0)),
            pl.BlockSpec((BB, C, T), lambda i: (i, 0, 0)),
            pl.BlockSpec((BB, K), lambda i: (i, 0)),
            pl.BlockSpec((BB, 5), lambda i: (i, 0)),
        ],
        out_shape=[
            jax.ShapeDtypeStruct((B, C, T), jnp.float32),
            jax.ShapeDtypeStruct((B, C, T), jnp.float32),
            jax.ShapeDtypeStruct((B, K), jnp.float32),
            jax.ShapeDtypeStruct((B, 5), jnp.int32),
        ],
        compiler_params=pltpu.CompilerParams(
            dimension_semantics=("parallel",)),
    )(inv2s, z_cont, idx2d,
      W_dec_cont, b_dec_cont.reshape(1, CT),
      W_dec_disc, b_dec_disc.reshape(1, CT))

    codebook_idxs = idxrow.reshape(B)
    return (codebook_idxs, z_cont, z_disc, nb, wgt, dist, xhc, xhd)
